# baseline (speedup 1.0000x reference)
import jax
import jax.numpy as jnp
from jax import lax
from jax.experimental import pallas as pl
from jax.experimental.pallas import tpu as pltpu
from jax.experimental.pallas import tpu_sc as plsc

N = 50000
E = 800000
NP = 50176
NBS = 3136
NB = 1568
NW = 32
EW = 25344
EPAD = EW * NW
KC = 128
NCHUNK = EW // KC
NPAIR = NCHUNK // 2
EPS = 1e-5


def _make_agg(C, mode):
  mesh = plsc.VectorSubcoreMesh(core_axis_name="c", subcore_axis_name="s")
  gather = mode != "count"

  out_t = jax.ShapeDtypeStruct((2, NP, 128), jnp.float32)

  if mode == "col4":
    NCHB = 33
    NBLK = NCHUNK // NCHB
    NTRIP = NCHB // 3
    scratch = [
        pltpu.VMEM((NCHB, KC), jnp.int32),
        pltpu.VMEM((NCHB, KC), jnp.int32),
        pltpu.VMEM((NCHB, KC), jnp.int32),
        pltpu.VMEM((NCHB, KC), jnp.int32),
        pltpu.VMEM((KC, C), jnp.float32),
        pltpu.VMEM((KC, C), jnp.float32),
        pltpu.VMEM((KC, C), jnp.float32),
        pltpu.VMEM_SHARED((NP, C), jnp.float32),
        pltpu.SemaphoreType.DMA,
        pltpu.SemaphoreType.DMA,
        pltpu.SemaphoreType.DMA,
        pltpu.SemaphoreType.DMA,
        pltpu.SemaphoreType.DMA,
    ]

    def body(tab, srcl4, dstl, zz, out, svi_a, svi_b, dvi_a, dvi_b,
             rows_a, rows_b, rows_c, acc, sem_a, sem_b, sem_c,
             sem_ia, sem_ib):
      cid = lax.axis_index("c")
      sid = lax.axis_index("s")
      wid = sid * 2 + cid
      rbase = sid * NBS
      svis = (svi_a, svi_b)
      dvis = (dvi_a, dvi_b)
      sems = (sem_ia, sem_ib)

      def idx_load(j, b, par):
        sl = pl.ds(b * NCHB, NCHB)
        pltpu.async_copy(srcl4.at[j, wid, sl], svis[par], sems[par])
        pltpu.async_copy(dstl.at[wid, sl], dvis[par], sems[par])

      def idx_wait(j, b, par):
        sl = pl.ds(b * NCHB, NCHB)
        pltpu.make_async_copy(srcl4.at[j, wid, sl], svis[par], sems[par]).wait()
        pltpu.make_async_copy(dstl.at[wid, sl], dvis[par], sems[par]).wait()

      for j in range(4):
        pltpu.sync_copy(zz, acc.at[pl.ds(rbase, NBS)])
        plsc.subcore_barrier()
        idx_load(j, 0, 0)
        for b in range(NBLK):
          par = b % 2
          svi, dvi = svis[par], dvis[par]
          idx_wait(j, b, par)
          if b < NBLK - 1:
            idx_load(j, b + 1, 1 - par)

          pltpu.async_copy(tab.at[svi.at[0]], rows_a, sem_a)
          pltpu.async_copy(tab.at[svi.at[1]], rows_b, sem_b)

          def trip(tt, carry):
            a, bb, cc = 3 * tt, 3 * tt + 1, 3 * tt + 2
            pltpu.async_copy(tab.at[svi.at[cc]], rows_c, sem_c)
            pltpu.make_async_copy(tab.at[svi.at[a]], rows_a, sem_a).wait()
            pltpu.sync_copy(rows_a, acc.at[dvi.at[a]], add=True)

            @pl.when(tt < NTRIP - 1)
            def _():
              pltpu.async_copy(tab.at[svi.at[a + 3]], rows_a, sem_a)

            pltpu.make_async_copy(tab.at[svi.at[bb]], rows_b, sem_b).wait()
            pltpu.sync_copy(rows_b, acc.at[dvi.at[bb]], add=True)

            @pl.when(tt < NTRIP - 1)
            def _():
              pltpu.async_copy(tab.at[svi.at[bb + 3]], rows_b, sem_b)

            pltpu.make_async_copy(tab.at[svi.at[cc]], rows_c, sem_c).wait()
            pltpu.sync_copy(rows_c, acc.at[dvi.at[cc]], add=True)
            return carry

          lax.fori_loop(0, NTRIP, trip, 0)

        plsc.subcore_barrier()
        pltpu.sync_copy(acc.at[pl.ds(rbase, NBS)],
                        out.at[cid, pl.ds(rbase, NBS), pl.ds(j * 32, 32)])
        plsc.subcore_barrier()

  else:
    scratch = [
        pltpu.VMEM((NCHUNK, KC), jnp.int32),
        pltpu.VMEM((NCHUNK, KC), jnp.int32),
        pltpu.VMEM((KC, C), jnp.float32),
        pltpu.VMEM((KC, C), jnp.float32),
        pltpu.VMEM_SHARED((NP, C), jnp.float32),
        pltpu.SemaphoreType.DMA,
        pltpu.SemaphoreType.DMA,
    ]

    def body(tab, srcl, dstl, zz, ones, out, svi2, dvi2,
             rows_a, rows_b, acc, sem_a, sem_b):
      cid = lax.axis_index("c")
      sid = lax.axis_index("s")
      wid = sid * 2 + cid
      rbase = sid * NBS

      pltpu.sync_copy(dstl.at[wid], dvi2)
      if gather:
        pltpu.sync_copy(srcl.at[wid], svi2)
      else:
        pltpu.sync_copy(ones, rows_a)

      pltpu.sync_copy(zz, acc.at[pl.ds(rbase, NBS)])
      plsc.subcore_barrier()

      if gather:
        pltpu.async_copy(tab.at[svi2.at[0]], rows_a, sem_a)

        def pair(t, carry):
          a, b = 2 * t, 2 * t + 1
          pltpu.async_copy(tab.at[svi2.at[b]], rows_b, sem_b)
          pltpu.make_async_copy(tab.at[svi2.at[a]], rows_a, sem_a).wait()
          pltpu.sync_copy(rows_a, acc.at[dvi2.at[a]], add=True)

          @pl.when(t < NPAIR - 1)
          def _():
            pltpu.async_copy(tab.at[svi2.at[a + 2]], rows_a, sem_a)

          pltpu.make_async_copy(tab.at[svi2.at[b]], rows_b, sem_b).wait()
          pltpu.sync_copy(rows_b, acc.at[dvi2.at[b]], add=True)
          return carry

        lax.fori_loop(0, NPAIR, pair, 0)
      else:
        def cnt_chunk(t, carry):
          pltpu.sync_copy(rows_a, acc.at[dvi2.at[t]], add=True)
          return carry
        lax.fori_loop(0, NCHUNK, cnt_chunk, 0)

      plsc.subcore_barrier()
      pltpu.sync_copy(acc.at[pl.ds(rbase, NBS)],
                      out.at[cid, pl.ds(rbase, NBS), pl.ds(0, C)])
      plsc.subcore_barrier()

  return pl.kernel(
      body,
      out_type=out_t,
      mesh=mesh,
      scratch_types=scratch,
      compiler_params=pltpu.CompilerParams(use_tc_tiling_on_sc=False),
  )


def _row_mask(pid):
  rid = pid * NB + lax.broadcasted_iota(jnp.int32, (NB, 1), 0)
  return rid < N


def _k1_body(cnt_ref, xp_ref, xs_ref, dinv_ref):
  deg = cnt_ref[0, :, 0:1] + cnt_ref[1, :, 0:1] + 1.0
  dinv = lax.rsqrt(deg)
  dinv_ref[...] = dinv
  xs_ref[...] = xp_ref[...] * dinv


def _k2_body(s_ref, xs_ref, dinv_ref, u_ref, usum_ref, um_ref):
  pid = pl.program_id(0)
  u = dinv_ref[...] * (s_ref[0, :, 0:16] + s_ref[1, :, 0:16] + xs_ref[...])
  u_ref[...] = u

  @pl.when(pid == 0)
  def _():
    usum_ref[...] = jnp.zeros_like(usum_ref)
    um_ref[...] = jnp.zeros_like(um_ref)

  usum_ref[...] += jnp.sum(u, axis=0, keepdims=True)
  um_ref[...] += jnp.dot(u.T, u, preferred_element_type=jnp.float32)


def _k3_body(u_ref, usum_ref, um_ref, w1_ref, b1_ref, g1_ref, be1_ref,
             dinv_ref, hs1_ref):
  pid = pl.program_id(0)
  w1 = w1_ref[...]
  mu = usum_ref[...] / N
  mean1 = jnp.dot(mu, w1) + b1_ref[...]
  cu = um_ref[...] / N - jnp.dot(mu.T, mu)
  var1 = jnp.sum(w1 * jnp.dot(cu, w1), axis=0, keepdims=True)
  s1 = g1_ref[...] * lax.rsqrt(var1 + EPS)
  weff = w1 * s1
  beff = (b1_ref[...] - mean1) * s1 + be1_ref[...]
  h1 = jnp.dot(u_ref[...], weff, preferred_element_type=jnp.float32) + beff
  h1 = jnp.where(h1 >= 0, h1, 0.1 * h1)
  hs1_ref[...] = jnp.where(_row_mask(pid), h1 * dinv_ref[...], 0.0)


def _k4_body(p_ref, hs1_ref, dinv_ref, w2_ref, b2_ref,
             agg2_ref, ssum_ref, ssq_ref):
  pid = pl.program_id(0)
  dinv = dinv_ref[...]
  gcn2 = jnp.zeros((NB, 256), jnp.float32)
  for j in range(4):
    sl = pl.ds(j * 32, 32)
    aj = dinv * (p_ref[0, :, sl] + p_ref[1, :, sl] + hs1_ref[:, sl])
    agg2_ref[:, sl] = aj
    gcn2 += jnp.dot(aj, w2_ref[j], preferred_element_type=jnp.float32)
  gcn2 += b2_ref[...]
  gm = jnp.where(_row_mask(pid), gcn2, 0.0)

  @pl.when(pid == 0)
  def _():
    ssum_ref[...] = jnp.zeros_like(ssum_ref)
    ssq_ref[...] = jnp.zeros_like(ssq_ref)

  ssum_ref[...] += jnp.sum(gm, axis=0, keepdims=True)
  ssq_ref[...] += jnp.sum(gm * gm, axis=0, keepdims=True)


def _k5_body(agg2_ref, ssum_ref, ssq_ref, w2_ref, b2_ref, g2_ref, be2_ref,
             w3_ref, dinv_ref, p16_ref):
  pid = pl.program_id(0)
  gcn2 = jnp.zeros((NB, 256), jnp.float32)
  for j in range(4):
    gcn2 += jnp.dot(agg2_ref[:, pl.ds(j * 32, 32)], w2_ref[j],
                    preferred_element_type=jnp.float32)
  gcn2 += b2_ref[...]
  m2 = ssum_ref[...] / N
  v2 = ssq_ref[...] / N - m2 * m2
  s2 = g2_ref[...] * lax.rsqrt(v2 + EPS)
  h2 = (gcn2 - m2) * s2 + be2_ref[...]
  h2 = jnp.where(h2 >= 0, h2, 0.3 * h2)
  p = jnp.dot(h2, w3_ref[...], preferred_element_type=jnp.float32)
  p = jnp.where(_row_mask(pid), p * dinv_ref[...], 0.0)
  colmask = (lax.broadcasted_iota(jnp.int32, (1, 16), 1) == 0).astype(
      jnp.float32)
  p16_ref[...] = jnp.broadcast_to(p, (NB, 16)) * colmask


def _k6_body(t_ref, p16_ref, dinv_ref, b3_ref, o_ref):
  g = dinv_ref[...] * (t_ref[0, :, 0:1] + t_ref[1, :, 0:1]
                       + p16_ref[:, 0:1]) + b3_ref[...]
  o_ref[...] = jax.nn.sigmoid(g)


def _bspec(shape, idx=None):
  if idx is None:
    idx = lambda i: tuple(0 for _ in shape)
  return pl.BlockSpec(shape, idx)


@jax.jit
def kernel(x, edge_index, W1, b1, g1, be1, W2, b2, g2, be2, W3, b3):
  f32 = jnp.float32
  src = edge_index[0]
  dst = edge_index[1]
  padi = jnp.full((EPAD - E,), N, jnp.int32)
  srcp = jnp.concatenate([src, padi]).reshape(NW, NCHUNK, KC)
  dstp = jnp.concatenate([dst, padi]).reshape(NW, NCHUNK, KC)

  xp = jnp.zeros((NP, 16), f32).at[:N, :3].set(x)
  w1p = jnp.zeros((16, 128), f32).at[:3].set(W1)
  w2r = W2.reshape(4, 32, 256)
  b1r, g1r, be1r = b1.reshape(1, 128), g1.reshape(1, 128), be1.reshape(1, 128)
  b2r, g2r, be2r = b2.reshape(1, 256), g2.reshape(1, 256), be2.reshape(1, 256)
  b3r = b3.reshape(1, 1)

  zz16 = jnp.zeros((NBS, 16), f32)
  zz32 = jnp.zeros((NBS, 32), f32)
  ones16 = jnp.ones((KC, 16), f32)
  dummy16 = jnp.zeros((8, 16), f32)

  cnt = _make_agg(16, "count")(dummy16, srcp, dstp, zz16, ones16)

  grid = (NP // NB,)
  xs, dinv = pl.pallas_call(
      _k1_body,
      grid=grid,
      in_specs=[_bspec((2, NB, 128), lambda i: (0, i, 0)),
                _bspec((NB, 16), lambda i: (i, 0))],
      out_specs=[_bspec((NB, 16), lambda i: (i, 0)),
                 _bspec((NB, 1), lambda i: (i, 0))],
      out_shape=[jax.ShapeDtypeStruct((NP, 16), f32),
                 jax.ShapeDtypeStruct((NP, 1), f32)],
  )(cnt, xp)

  s = _make_agg(16, "plain")(xs, srcp, dstp, zz16, ones16)

  u, usum, um = pl.pallas_call(
      _k2_body,
      grid=grid,
      in_specs=[_bspec((2, NB, 128), lambda i: (0, i, 0)),
                _bspec((NB, 16), lambda i: (i, 0)),
                _bspec((NB, 1), lambda i: (i, 0))],
      out_specs=[_bspec((NB, 16), lambda i: (i, 0)),
                 _bspec((1, 16)),
                 _bspec((16, 16))],
      out_shape=[jax.ShapeDtypeStruct((NP, 16), f32),
                 jax.ShapeDtypeStruct((1, 16), f32),
                 jax.ShapeDtypeStruct((16, 16), f32)],
  )(s, xs, dinv)

  hs1 = pl.pallas_call(
      _k3_body,
      grid=grid,
      in_specs=[_bspec((NB, 16), lambda i: (i, 0)),
                _bspec((1, 16)), _bspec((16, 16)), _bspec((16, 128)),
                _bspec((1, 128)), _bspec((1, 128)), _bspec((1, 128)),
                _bspec((NB, 1), lambda i: (i, 0))],
      out_specs=_bspec((NB, 128), lambda i: (i, 0)),
      out_shape=jax.ShapeDtypeStruct((NP, 128), f32),
  )(u, usum, um, w1p, b1r, g1r, be1r, dinv)

  hs1_flat = hs1.reshape(4 * NP, 32)
  src4 = (srcp * 4)[None] + jnp.arange(4, dtype=jnp.int32)[:, None, None, None]
  p = _make_agg(32, "col4")(hs1_flat, src4, dstp, zz32)

  agg2, ssum, ssq = pl.pallas_call(
      _k4_body,
      grid=grid,
      in_specs=[_bspec((2, NB, 128), lambda i: (0, i, 0)),
                _bspec((NB, 128), lambda i: (i, 0)),
                _bspec((NB, 1), lambda i: (i, 0)),
                _bspec((4, 32, 256)), _bspec((1, 256))],
      out_specs=[_bspec((NB, 128), lambda i: (i, 0)),
                 _bspec((1, 256)), _bspec((1, 256))],
      out_shape=[jax.ShapeDtypeStruct((NP, 128), f32),
                 jax.ShapeDtypeStruct((1, 256), f32),
                 jax.ShapeDtypeStruct((1, 256), f32)],
  )(p, hs1, dinv, w2r, b2r)

  p16 = pl.pallas_call(
      _k5_body,
      grid=grid,
      in_specs=[_bspec((NB, 128), lambda i: (i, 0)),
                _bspec((1, 256)), _bspec((1, 256)),
                _bspec((4, 32, 256)), _bspec((1, 256)), _bspec((1, 256)),
                _bspec((1, 256)), _bspec((256, 1)),
                _bspec((NB, 1), lambda i: (i, 0))],
      out_specs=_bspec((NB, 16), lambda i: (i, 0)),
      out_shape=jax.ShapeDtypeStruct((NP, 16), f32),
  )(agg2, ssum, ssq, w2r, b2r, g2r, be2r, W3, dinv)

  t = _make_agg(16, "plain")(p16, srcp, dstp, zz16, ones16)

  o = pl.pallas_call(
      _k6_body,
      grid=grid,
      in_specs=[_bspec((2, NB, 128), lambda i: (0, i, 0)),
                _bspec((NB, 16), lambda i: (i, 0)),
                _bspec((NB, 1), lambda i: (i, 0)),
                _bspec((1, 1))],
      out_specs=_bspec((NB, 1), lambda i: (i, 0)),
      out_shape=jax.ShapeDtypeStruct((NP, 1), f32),
  )(t, p16, dinv, b3r)

  return o[:N]


# speedup vs baseline: 1.1787x; 1.1787x over previous
import jax
import jax.numpy as jnp
from jax import lax
from jax.experimental import pallas as pl
from jax.experimental.pallas import tpu as pltpu
from jax.experimental.pallas import tpu_sc as plsc

N = 50000
E = 800000
NP = 50176
NBS = 3136
NB = 1568
NW = 32
EW = 25088
EPAD = EW * NW
KC = 128
NCHUNK = EW // KC
NPAIR = NCHUNK // 2
EPS = 1e-5


def _make_agg(C, mode):
  mesh = plsc.VectorSubcoreMesh(core_axis_name="c", subcore_axis_name="s")
  gather = mode != "count"

  if mode == "col4":
    out_t = jax.ShapeDtypeStruct((NP, 128), jnp.float32)
  else:
    out_t = jax.ShapeDtypeStruct((2, NP, C), jnp.float32)

  if mode == "col4":
    NCHB = 28
    NCH2 = 2 * NCHUNK
    NBLK = NCH2 // NCHB
    NPAIRB = NCHB // 2
    scratch = [
        pltpu.VMEM((NCHB, KC), jnp.int32),
        pltpu.VMEM((NCHB, KC), jnp.int32),
        pltpu.VMEM((NCHB, KC), jnp.int32),
        pltpu.VMEM((NCHB, KC), jnp.int32),
        pltpu.VMEM((KC, C), jnp.float32),
        pltpu.VMEM((KC, C), jnp.float32),
        pltpu.VMEM_SHARED((NP, C), jnp.float32),
        pltpu.SemaphoreType.DMA,
        pltpu.SemaphoreType.DMA,
        pltpu.SemaphoreType.DMA,
        pltpu.SemaphoreType.DMA,
    ]

    def body(tab, srcl4, dstl, zz, out, svi_a, svi_b, dvi_a, dvi_b,
             rows_a, rows_b, acc, sem_a, sem_b, sem_ia, sem_ib):
      cid = lax.axis_index("c")
      sid = lax.axis_index("s")
      rbase = sid * NBS
      svis = (svi_a, svi_b)
      dvis = (dvi_a, dvi_b)
      sems = (sem_ia, sem_ib)

      def idx_load(j, b, par):
        sl = pl.ds(b * NCHB, NCHB)
        pltpu.async_copy(srcl4.at[j, sid, sl], svis[par], sems[par])
        pltpu.async_copy(dstl.at[sid, sl], dvis[par], sems[par])

      def idx_wait(j, b, par):
        sl = pl.ds(b * NCHB, NCHB)
        pltpu.make_async_copy(srcl4.at[j, sid, sl], svis[par], sems[par]).wait()
        pltpu.make_async_copy(dstl.at[sid, sl], dvis[par], sems[par]).wait()

      for jl in range(2):
        j = cid * 2 + jl
        pltpu.sync_copy(zz, acc.at[pl.ds(rbase, NBS)])
        plsc.subcore_barrier()
        idx_load(j, 0, 0)
        for b in range(NBLK):
          par = b % 2
          svi, dvi = svis[par], dvis[par]
          idx_wait(j, b, par)
          if b < NBLK - 1:
            idx_load(j, b + 1, 1 - par)

          pltpu.async_copy(tab.at[svi.at[0]], rows_a, sem_a)

          def pair(tp, carry):
            a, bb = 2 * tp, 2 * tp + 1
            pltpu.async_copy(tab.at[svi.at[bb]], rows_b, sem_b)
            pltpu.make_async_copy(tab.at[svi.at[a]], rows_a, sem_a).wait()
            pltpu.sync_copy(rows_a, acc.at[dvi.at[a]], add=True)

            @pl.when(tp < NPAIRB - 1)
            def _():
              pltpu.async_copy(tab.at[svi.at[a + 2]], rows_a, sem_a)

            pltpu.make_async_copy(tab.at[svi.at[bb]], rows_b, sem_b).wait()
            pltpu.sync_copy(rows_b, acc.at[dvi.at[bb]], add=True)
            return carry

          lax.fori_loop(0, NPAIRB, pair, 0)

        plsc.subcore_barrier()
        pltpu.sync_copy(acc.at[pl.ds(rbase, NBS)],
                        out.at[pl.ds(rbase, NBS), pl.ds(j * 32, 32)])
        plsc.subcore_barrier()

  else:
    scratch = [
        pltpu.VMEM((NCHUNK, KC), jnp.int32),
        pltpu.VMEM((NCHUNK, KC), jnp.int32),
        pltpu.VMEM((KC, C), jnp.float32),
        pltpu.VMEM((KC, C), jnp.float32),
        pltpu.VMEM_SHARED((NP, C), jnp.float32),
        pltpu.SemaphoreType.DMA,
        pltpu.SemaphoreType.DMA,
    ]

    def body(tab, srcl, dstl, zz, ones, out, svi2, dvi2,
             rows_a, rows_b, acc, sem_a, sem_b):
      cid = lax.axis_index("c")
      sid = lax.axis_index("s")
      wid = sid * 2 + cid
      rbase = sid * NBS

      pltpu.sync_copy(dstl.at[wid], dvi2)
      if gather:
        pltpu.sync_copy(srcl.at[wid], svi2)
      else:
        pltpu.sync_copy(ones, rows_a)

      pltpu.sync_copy(zz, acc.at[pl.ds(rbase, NBS)])
      plsc.subcore_barrier()

      if gather:
        pltpu.async_copy(tab.at[svi2.at[0]], rows_a, sem_a)

        def pair(t, carry):
          a, b = 2 * t, 2 * t + 1
          pltpu.async_copy(tab.at[svi2.at[b]], rows_b, sem_b)
          pltpu.make_async_copy(tab.at[svi2.at[a]], rows_a, sem_a).wait()
          pltpu.sync_copy(rows_a, acc.at[dvi2.at[a]], add=True)

          @pl.when(t < NPAIR - 1)
          def _():
            pltpu.async_copy(tab.at[svi2.at[a + 2]], rows_a, sem_a)

          pltpu.make_async_copy(tab.at[svi2.at[b]], rows_b, sem_b).wait()
          pltpu.sync_copy(rows_b, acc.at[dvi2.at[b]], add=True)
          return carry

        lax.fori_loop(0, NPAIR, pair, 0)
      else:
        def cnt_chunk(t, carry):
          pltpu.sync_copy(rows_a, acc.at[dvi2.at[t]], add=True)
          return carry
        lax.fori_loop(0, NCHUNK, cnt_chunk, 0)

      plsc.subcore_barrier()
      pltpu.sync_copy(acc.at[pl.ds(rbase, NBS)],
                      out.at[cid, pl.ds(rbase, NBS)])
      plsc.subcore_barrier()

  return pl.kernel(
      body,
      out_type=out_t,
      mesh=mesh,
      scratch_types=scratch,
      compiler_params=pltpu.CompilerParams(use_tc_tiling_on_sc=False),
  )


def _row_mask(pid):
  rid = pid * NB + lax.broadcasted_iota(jnp.int32, (NB, 1), 0)
  return rid < N


def _k1_body(cnt_ref, xp_ref, xs_ref, dinv_ref):
  deg = cnt_ref[0, :, 0:1] + cnt_ref[1, :, 0:1] + 1.0
  dinv = lax.rsqrt(deg)
  dinv_ref[...] = dinv
  xs_ref[...] = xp_ref[...] * dinv


def _k2_body(s_ref, xs_ref, dinv_ref, u_ref, usum_ref, um_ref):
  pid = pl.program_id(0)
  u = dinv_ref[...] * (s_ref[0] + s_ref[1] + xs_ref[...])
  u_ref[...] = u

  @pl.when(pid == 0)
  def _():
    usum_ref[...] = jnp.zeros_like(usum_ref)
    um_ref[...] = jnp.zeros_like(um_ref)

  usum_ref[...] += jnp.sum(u, axis=0, keepdims=True)
  um_ref[...] += jnp.dot(u.T, u, preferred_element_type=jnp.float32)


def _k3_body(u_ref, usum_ref, um_ref, w1_ref, b1_ref, g1_ref, be1_ref,
             dinv_ref, hs1_ref):
  pid = pl.program_id(0)
  w1 = w1_ref[...]
  mu = usum_ref[...] / N
  mean1 = jnp.dot(mu, w1) + b1_ref[...]
  cu = um_ref[...] / N - jnp.dot(mu.T, mu)
  var1 = jnp.sum(w1 * jnp.dot(cu, w1), axis=0, keepdims=True)
  s1 = g1_ref[...] * lax.rsqrt(var1 + EPS)
  weff = w1 * s1
  beff = (b1_ref[...] - mean1) * s1 + be1_ref[...]
  h1 = jnp.dot(u_ref[...], weff, preferred_element_type=jnp.float32) + beff
  h1 = jnp.where(h1 >= 0, h1, 0.1 * h1)
  hs1_ref[...] = jnp.where(_row_mask(pid), h1 * dinv_ref[...], 0.0)


def _k4_body(p_ref, hs1_ref, dinv_ref, w2_ref, b2_ref,
             agg2_ref, ssum_ref, ssq_ref):
  pid = pl.program_id(0)
  dinv = dinv_ref[...]
  gcn2 = jnp.zeros((NB, 256), jnp.float32)
  for j in range(4):
    sl = pl.ds(j * 32, 32)
    aj = dinv * (p_ref[:, sl] + hs1_ref[:, sl])
    agg2_ref[:, sl] = aj
    gcn2 += jnp.dot(aj, w2_ref[j], preferred_element_type=jnp.float32)
  gcn2 += b2_ref[...]
  gm = jnp.where(_row_mask(pid), gcn2, 0.0)

  @pl.when(pid == 0)
  def _():
    ssum_ref[...] = jnp.zeros_like(ssum_ref)
    ssq_ref[...] = jnp.zeros_like(ssq_ref)

  ssum_ref[...] += jnp.sum(gm, axis=0, keepdims=True)
  ssq_ref[...] += jnp.sum(gm * gm, axis=0, keepdims=True)


def _k5_body(agg2_ref, ssum_ref, ssq_ref, w2_ref, b2_ref, g2_ref, be2_ref,
             w3_ref, dinv_ref, p16_ref):
  pid = pl.program_id(0)
  gcn2 = jnp.zeros((NB, 256), jnp.float32)
  for j in range(4):
    gcn2 += jnp.dot(agg2_ref[:, pl.ds(j * 32, 32)], w2_ref[j],
                    preferred_element_type=jnp.float32)
  gcn2 += b2_ref[...]
  m2 = ssum_ref[...] / N
  v2 = ssq_ref[...] / N - m2 * m2
  s2 = g2_ref[...] * lax.rsqrt(v2 + EPS)
  h2 = (gcn2 - m2) * s2 + be2_ref[...]
  h2 = jnp.where(h2 >= 0, h2, 0.3 * h2)
  p = jnp.dot(h2, w3_ref[...], preferred_element_type=jnp.float32)
  p = jnp.where(_row_mask(pid), p * dinv_ref[...], 0.0)
  colmask = (lax.broadcasted_iota(jnp.int32, (1, 16), 1) == 0).astype(
      jnp.float32)
  p16_ref[...] = jnp.broadcast_to(p, (NB, 16)) * colmask


def _k6_body(t_ref, p16_ref, dinv_ref, b3_ref, o_ref):
  g = dinv_ref[...] * (t_ref[0, :, 0:1] + t_ref[1, :, 0:1]
                       + p16_ref[:, 0:1]) + b3_ref[...]
  o_ref[...] = jax.nn.sigmoid(g)


def _bspec(shape, idx=None):
  if idx is None:
    idx = lambda i: tuple(0 for _ in shape)
  return pl.BlockSpec(shape, idx)


@jax.jit
def kernel(x, edge_index, W1, b1, g1, be1, W2, b2, g2, be2, W3, b3):
  f32 = jnp.float32
  src = edge_index[0]
  dst = edge_index[1]
  padi = jnp.full((EPAD - E,), N, jnp.int32)
  srcp = jnp.concatenate([src, padi]).reshape(NW, NCHUNK, KC)
  dstp = jnp.concatenate([dst, padi]).reshape(NW, NCHUNK, KC)

  xp = jnp.zeros((NP, 16), f32).at[:N, :3].set(x)
  w1p = jnp.zeros((16, 128), f32).at[:3].set(W1)
  w2r = W2.reshape(4, 32, 256)
  b1r, g1r, be1r = b1.reshape(1, 128), g1.reshape(1, 128), be1.reshape(1, 128)
  b2r, g2r, be2r = b2.reshape(1, 256), g2.reshape(1, 256), be2.reshape(1, 256)
  b3r = b3.reshape(1, 1)

  zz16 = jnp.zeros((NBS, 16), f32)
  zz32 = jnp.zeros((NBS, 32), f32)
  ones16 = jnp.ones((KC, 16), f32)
  dummy16 = jnp.zeros((8, 16), f32)

  cnt = _make_agg(16, "count")(dummy16, srcp, dstp, zz16, ones16)

  grid = (NP // NB,)
  xs, dinv = pl.pallas_call(
      _k1_body,
      grid=grid,
      in_specs=[_bspec((2, NB, 16), lambda i: (0, i, 0)),
                _bspec((NB, 16), lambda i: (i, 0))],
      out_specs=[_bspec((NB, 16), lambda i: (i, 0)),
                 _bspec((NB, 1), lambda i: (i, 0))],
      out_shape=[jax.ShapeDtypeStruct((NP, 16), f32),
                 jax.ShapeDtypeStruct((NP, 1), f32)],
  )(cnt, xp)

  s = _make_agg(16, "plain")(xs, srcp, dstp, zz16, ones16)

  u, usum, um = pl.pallas_call(
      _k2_body,
      grid=grid,
      in_specs=[_bspec((2, NB, 16), lambda i: (0, i, 0)),
                _bspec((NB, 16), lambda i: (i, 0)),
                _bspec((NB, 1), lambda i: (i, 0))],
      out_specs=[_bspec((NB, 16), lambda i: (i, 0)),
                 _bspec((1, 16)),
                 _bspec((16, 16))],
      out_shape=[jax.ShapeDtypeStruct((NP, 16), f32),
                 jax.ShapeDtypeStruct((1, 16), f32),
                 jax.ShapeDtypeStruct((16, 16), f32)],
  )(s, xs, dinv)

  hs1 = pl.pallas_call(
      _k3_body,
      grid=grid,
      in_specs=[_bspec((NB, 16), lambda i: (i, 0)),
                _bspec((1, 16)), _bspec((16, 16)), _bspec((16, 128)),
                _bspec((1, 128)), _bspec((1, 128)), _bspec((1, 128)),
                _bspec((NB, 1), lambda i: (i, 0))],
      out_specs=_bspec((NB, 128), lambda i: (i, 0)),
      out_shape=jax.ShapeDtypeStruct((NP, 128), f32),
  )(u, usum, um, w1p, b1r, g1r, be1r, dinv)

  hs1_flat = hs1.reshape(4 * NP, 32)
  src4 = (srcp * 4)[None] + jnp.arange(4, dtype=jnp.int32)[:, None, None, None]
  src4t = src4.reshape(4, 16, 2 * NCHUNK, KC)
  dstt = dstp.reshape(16, 2 * NCHUNK, KC)
  p = _make_agg(32, "col4")(hs1_flat, src4t, dstt, zz32)

  agg2, ssum, ssq = pl.pallas_call(
      _k4_body,
      grid=grid,
      in_specs=[_bspec((NB, 128), lambda i: (i, 0)),
                _bspec((NB, 128), lambda i: (i, 0)),
                _bspec((NB, 1), lambda i: (i, 0)),
                _bspec((4, 32, 256)), _bspec((1, 256))],
      out_specs=[_bspec((NB, 128), lambda i: (i, 0)),
                 _bspec((1, 256)), _bspec((1, 256))],
      out_shape=[jax.ShapeDtypeStruct((NP, 128), f32),
                 jax.ShapeDtypeStruct((1, 256), f32),
                 jax.ShapeDtypeStruct((1, 256), f32)],
  )(p, hs1, dinv, w2r, b2r)

  p16 = pl.pallas_call(
      _k5_body,
      grid=grid,
      in_specs=[_bspec((NB, 128), lambda i: (i, 0)),
                _bspec((1, 256)), _bspec((1, 256)),
                _bspec((4, 32, 256)), _bspec((1, 256)), _bspec((1, 256)),
                _bspec((1, 256)), _bspec((256, 1)),
                _bspec((NB, 1), lambda i: (i, 0))],
      out_specs=_bspec((NB, 16), lambda i: (i, 0)),
      out_shape=jax.ShapeDtypeStruct((NP, 16), f32),
  )(agg2, ssum, ssq, w2r, b2r, g2r, be2r, W3, dinv)

  t = _make_agg(16, "plain")(p16, srcp, dstp, zz16, ones16)

  o = pl.pallas_call(
      _k6_body,
      grid=grid,
      in_specs=[_bspec((2, NB, 16), lambda i: (0, i, 0)),
                _bspec((NB, 16), lambda i: (i, 0)),
                _bspec((NB, 1), lambda i: (i, 0)),
                _bspec((1, 1))],
      out_specs=_bspec((NB, 1), lambda i: (i, 0)),
      out_shape=jax.ShapeDtypeStruct((NP, 1), f32),
  )(t, p16, dinv, b3r)

  return o[:N]


# speedup vs baseline: 1.2033x; 1.0208x over previous
import jax
import jax.numpy as jnp
from jax import lax
from jax.experimental import pallas as pl
from jax.experimental.pallas import tpu as pltpu
from jax.experimental.pallas import tpu_sc as plsc

N = 50000
E = 800000
NP = 50176
NBS = 3136
NB = 1568
NW = 32
EW = 25088
EPAD = EW * NW
KC = 128
NCHUNK = EW // KC
NPAIR = NCHUNK // 2
EPS = 1e-5


def _make_agg(C, mode):
  mesh = plsc.VectorSubcoreMesh(core_axis_name="c", subcore_axis_name="s")
  gather = mode != "count"

  if mode == "col4":
    out_t = jax.ShapeDtypeStruct((NP, 128), jnp.float32)
  else:
    out_t = jax.ShapeDtypeStruct((2, NP, C), jnp.float32)

  if mode == "col4":
    NCHB = 28
    NCH2 = 2 * NCHUNK
    NBLK = NCH2 // NCHB
    NPAIRB = NCHB // 2
    scratch = [
        pltpu.VMEM((NCHB, KC), jnp.int32),
        pltpu.VMEM((NCHB, KC), jnp.int32),
        pltpu.VMEM((NCHB, KC), jnp.int32),
        pltpu.VMEM((NCHB, KC), jnp.int32),
        pltpu.VMEM((KC, C), jnp.float32),
        pltpu.VMEM((KC, C), jnp.float32),
        pltpu.VMEM_SHARED((NP, C), jnp.float32),
        pltpu.SemaphoreType.DMA,
        pltpu.SemaphoreType.DMA,
        pltpu.SemaphoreType.DMA,
        pltpu.SemaphoreType.DMA,
    ]

    def body(tab, srcl4, dstl, zz, out, svi_a, svi_b, dvi_a, dvi_b,
             rows_a, rows_b, acc, sem_a, sem_b, sem_ia, sem_ib):
      cid = lax.axis_index("c")
      sid = lax.axis_index("s")
      rbase = sid * NBS
      svis = (svi_a, svi_b)
      dvis = (dvi_a, dvi_b)
      sems = (sem_ia, sem_ib)

      def idx_load(j, b, par):
        sl = pl.ds(b * NCHB, NCHB)
        pltpu.async_copy(srcl4.at[j, sid, sl], svis[par], sems[par])
        pltpu.async_copy(dstl.at[sid, sl], dvis[par], sems[par])

      def idx_wait(j, b, par):
        sl = pl.ds(b * NCHB, NCHB)
        pltpu.make_async_copy(srcl4.at[j, sid, sl], svis[par], sems[par]).wait()
        pltpu.make_async_copy(dstl.at[sid, sl], dvis[par], sems[par]).wait()

      for jl in range(2):
        j = cid * 2 + jl
        pltpu.sync_copy(zz, acc.at[pl.ds(rbase, NBS)])
        plsc.subcore_barrier()
        idx_load(j, 0, 0)
        for b in range(NBLK):
          par = b % 2
          svi, dvi = svis[par], dvis[par]
          idx_wait(j, b, par)
          if b < NBLK - 1:
            idx_load(j, b + 1, 1 - par)

          pltpu.async_copy(tab.at[svi.at[0]], rows_a, sem_a)

          def pair(tp, carry):
            a, bb = 2 * tp, 2 * tp + 1
            pltpu.async_copy(tab.at[svi.at[bb]], rows_b, sem_b)
            pltpu.make_async_copy(tab.at[svi.at[a]], rows_a, sem_a).wait()
            pltpu.sync_copy(rows_a, acc.at[dvi.at[a]], add=True)

            @pl.when(tp < NPAIRB - 1)
            def _():
              pltpu.async_copy(tab.at[svi.at[a + 2]], rows_a, sem_a)

            pltpu.make_async_copy(tab.at[svi.at[bb]], rows_b, sem_b).wait()
            pltpu.sync_copy(rows_b, acc.at[dvi.at[bb]], add=True)
            return carry

          lax.fori_loop(0, NPAIRB, pair, 0)

        plsc.subcore_barrier()
        pltpu.sync_copy(acc.at[pl.ds(rbase, NBS)],
                        out.at[pl.ds(rbase, NBS), pl.ds(j * 32, 32)])
        plsc.subcore_barrier()

  else:
    scratch = [
        pltpu.VMEM((NCHUNK, KC), jnp.int32),
        pltpu.VMEM((NCHUNK, KC), jnp.int32),
        pltpu.VMEM((KC, C), jnp.float32),
        pltpu.VMEM((KC, C), jnp.float32),
        pltpu.VMEM_SHARED((NP, C), jnp.float32),
        pltpu.SemaphoreType.DMA,
        pltpu.SemaphoreType.DMA,
    ]

    def body(tab, srcl, dstl, zz, ones, out, svi2, dvi2,
             rows_a, rows_b, acc, sem_a, sem_b):
      cid = lax.axis_index("c")
      sid = lax.axis_index("s")
      wid = sid * 2 + cid
      rbase = sid * NBS

      pltpu.sync_copy(dstl.at[wid], dvi2)
      if gather:
        pltpu.sync_copy(srcl.at[wid], svi2)
      else:
        pltpu.sync_copy(ones, rows_a)

      pltpu.sync_copy(zz, acc.at[pl.ds(rbase, NBS)])
      plsc.subcore_barrier()

      if gather:
        pltpu.async_copy(tab.at[svi2.at[0]], rows_a, sem_a)

        def pair(t, carry):
          a, b = 2 * t, 2 * t + 1
          pltpu.async_copy(tab.at[svi2.at[b]], rows_b, sem_b)
          pltpu.make_async_copy(tab.at[svi2.at[a]], rows_a, sem_a).wait()
          pltpu.sync_copy(rows_a, acc.at[dvi2.at[a]], add=True)

          @pl.when(t < NPAIR - 1)
          def _():
            pltpu.async_copy(tab.at[svi2.at[a + 2]], rows_a, sem_a)

          pltpu.make_async_copy(tab.at[svi2.at[b]], rows_b, sem_b).wait()
          pltpu.sync_copy(rows_b, acc.at[dvi2.at[b]], add=True)
          return carry

        lax.fori_loop(0, NPAIR, pair, 0)
      else:
        def cnt_chunk(t, carry):
          pltpu.sync_copy(rows_a, acc.at[dvi2.at[t]], add=True)
          return carry
        lax.fori_loop(0, NCHUNK, cnt_chunk, 0)

      plsc.subcore_barrier()
      pltpu.sync_copy(acc.at[pl.ds(rbase, NBS)],
                      out.at[cid, pl.ds(rbase, NBS)])
      plsc.subcore_barrier()

  return pl.kernel(
      body,
      out_type=out_t,
      mesh=mesh,
      scratch_types=scratch,
      compiler_params=pltpu.CompilerParams(use_tc_tiling_on_sc=False),
  )


def _row_mask(pid):
  rid = pid * NB + lax.broadcasted_iota(jnp.int32, (NB, 1), 0)
  return rid < N


def _k1_body(cnt_ref, xp_ref, xs_ref, dinv_ref):
  deg = cnt_ref[0, :, 0:1] + cnt_ref[1, :, 0:1] + 1.0
  dinv = lax.rsqrt(deg)
  dinv_ref[...] = dinv
  col3 = lax.broadcasted_iota(jnp.int32, (1, 16), 1) == 3
  xs_ref[...] = jnp.where(col3, dinv, xp_ref[...] * dinv)


def _k2_body(s_ref, xs_ref, u_ref, usum_ref, um_ref):
  pid = pl.program_id(0)
  xs = xs_ref[...]
  dinv = xs[:, 3:4]
  col3 = lax.broadcasted_iota(jnp.int32, (1, 16), 1) == 3
  u = jnp.where(col3, dinv, dinv * (s_ref[0] + s_ref[1] + xs))
  u_ref[...] = u

  @pl.when(pid == 0)
  def _():
    usum_ref[...] = jnp.zeros_like(usum_ref)
    um_ref[...] = jnp.zeros_like(um_ref)

  usum_ref[...] += jnp.sum(u, axis=0, keepdims=True)
  um_ref[...] += jnp.dot(u.T, u, preferred_element_type=jnp.float32)


def _k3_body(u_ref, usum_ref, um_ref, w1_ref, b1_ref, g1_ref, be1_ref,
             hs1_ref):
  pid = pl.program_id(0)
  w1 = w1_ref[...]
  mu = usum_ref[...] / N
  mean1 = jnp.dot(mu, w1) + b1_ref[...]
  cu = um_ref[...] / N - jnp.dot(mu.T, mu)
  var1 = jnp.sum(w1 * jnp.dot(cu, w1), axis=0, keepdims=True)
  s1 = g1_ref[...] * lax.rsqrt(var1 + EPS)
  weff = w1 * s1
  beff = (b1_ref[...] - mean1) * s1 + be1_ref[...]
  u = u_ref[...]
  dinv = u[:, 3:4]
  h1 = jnp.dot(u, weff, preferred_element_type=jnp.float32) + beff
  h1 = jnp.where(h1 >= 0, h1, 0.1 * h1)
  hs1_ref[...] = jnp.where(_row_mask(pid), h1 * dinv, 0.0)


def _k4_body(p_ref, hs1_ref, dinv_ref, w2_ref, b2_ref,
             agg2_ref, ssum_ref, ssq_ref):
  pid = pl.program_id(0)
  dinv = dinv_ref[...]
  gcn2 = jnp.zeros((NB, 256), jnp.float32)
  for j in range(4):
    sl = pl.ds(j * 32, 32)
    aj = dinv * (p_ref[:, sl] + hs1_ref[:, sl])
    agg2_ref[:, sl] = aj
    gcn2 += jnp.dot(aj, w2_ref[j], preferred_element_type=jnp.float32)
  gcn2 += b2_ref[...]
  gm = jnp.where(_row_mask(pid), gcn2, 0.0)

  @pl.when(pid == 0)
  def _():
    ssum_ref[...] = jnp.zeros_like(ssum_ref)
    ssq_ref[...] = jnp.zeros_like(ssq_ref)

  ssum_ref[...] += jnp.sum(gm, axis=0, keepdims=True)
  ssq_ref[...] += jnp.sum(gm * gm, axis=0, keepdims=True)


def _k5_body(agg2_ref, ssum_ref, ssq_ref, w2_ref, b2_ref, g2_ref, be2_ref,
             w3_ref, dinv_ref, p16_ref):
  pid = pl.program_id(0)
  gcn2 = jnp.zeros((NB, 256), jnp.float32)
  for j in range(4):
    gcn2 += jnp.dot(agg2_ref[:, pl.ds(j * 32, 32)], w2_ref[j],
                    preferred_element_type=jnp.float32)
  gcn2 += b2_ref[...]
  m2 = ssum_ref[...] / N
  v2 = ssq_ref[...] / N - m2 * m2
  s2 = g2_ref[...] * lax.rsqrt(v2 + EPS)
  h2 = (gcn2 - m2) * s2 + be2_ref[...]
  h2 = jnp.where(h2 >= 0, h2, 0.3 * h2)
  p = jnp.dot(h2, w3_ref[...], preferred_element_type=jnp.float32)
  p = jnp.where(_row_mask(pid), p * dinv_ref[...], 0.0)
  colmask = (lax.broadcasted_iota(jnp.int32, (1, 16), 1) == 0).astype(
      jnp.float32)
  p16_ref[...] = jnp.broadcast_to(p, (NB, 16)) * colmask


def _k6_body(t_ref, p16_ref, dinv_ref, b3_ref, o_ref):
  g = dinv_ref[...] * (t_ref[0, :, 0:1] + t_ref[1, :, 0:1]
                       + p16_ref[:, 0:1]) + b3_ref[...]
  o_ref[...] = jax.nn.sigmoid(g)


def _bspec(shape, idx=None):
  if idx is None:
    idx = lambda i: tuple(0 for _ in shape)
  return pl.BlockSpec(shape, idx)


@jax.jit
def kernel(x, edge_index, W1, b1, g1, be1, W2, b2, g2, be2, W3, b3):
  f32 = jnp.float32
  src = edge_index[0]
  dst = edge_index[1]
  padi = jnp.full((EPAD - E,), N, jnp.int32)
  srcp = jnp.concatenate([src, padi]).reshape(NW, NCHUNK, KC)
  dstp = jnp.concatenate([dst, padi]).reshape(NW, NCHUNK, KC)

  xp = jnp.zeros((NP, 16), f32).at[:N, :3].set(x)
  w1p = jnp.zeros((16, 128), f32).at[:3].set(W1)
  w2r = W2.reshape(4, 32, 256)
  b1r, g1r, be1r = b1.reshape(1, 128), g1.reshape(1, 128), be1.reshape(1, 128)
  b2r, g2r, be2r = b2.reshape(1, 256), g2.reshape(1, 256), be2.reshape(1, 256)
  b3r = b3.reshape(1, 1)

  zz16 = jnp.zeros((NBS, 16), f32)
  zz32 = jnp.zeros((NBS, 32), f32)
  ones16 = jnp.ones((KC, 16), f32)
  dummy16 = jnp.zeros((8, 16), f32)

  cnt = _make_agg(16, "count")(dummy16, srcp, dstp, zz16, ones16)

  grid = (NP // NB,)
  xs, dinv = pl.pallas_call(
      _k1_body,
      grid=grid,
      in_specs=[_bspec((2, NB, 16), lambda i: (0, i, 0)),
                _bspec((NB, 16), lambda i: (i, 0))],
      out_specs=[_bspec((NB, 16), lambda i: (i, 0)),
                 _bspec((NB, 1), lambda i: (i, 0))],
      out_shape=[jax.ShapeDtypeStruct((NP, 16), f32),
                 jax.ShapeDtypeStruct((NP, 1), f32)],
  )(cnt, xp)

  s = _make_agg(16, "plain")(xs, srcp, dstp, zz16, ones16)

  u, usum, um = pl.pallas_call(
      _k2_body,
      grid=grid,
      in_specs=[_bspec((2, NB, 16), lambda i: (0, i, 0)),
                _bspec((NB, 16), lambda i: (i, 0))],
      out_specs=[_bspec((NB, 16), lambda i: (i, 0)),
                 _bspec((1, 16)),
                 _bspec((16, 16))],
      out_shape=[jax.ShapeDtypeStruct((NP, 16), f32),
                 jax.ShapeDtypeStruct((1, 16), f32),
                 jax.ShapeDtypeStruct((16, 16), f32)],
  )(s, xs)

  hs1 = pl.pallas_call(
      _k3_body,
      grid=grid,
      in_specs=[_bspec((NB, 16), lambda i: (i, 0)),
                _bspec((1, 16)), _bspec((16, 16)), _bspec((16, 128)),
                _bspec((1, 128)), _bspec((1, 128)), _bspec((1, 128))],
      out_specs=_bspec((NB, 128), lambda i: (i, 0)),
      out_shape=jax.ShapeDtypeStruct((NP, 128), f32),
  )(u, usum, um, w1p, b1r, g1r, be1r)

  hs1_flat = hs1.reshape(4 * NP, 32)
  src4 = (srcp * 4)[None] + jnp.arange(4, dtype=jnp.int32)[:, None, None, None]
  src4t = src4.reshape(4, 16, 2 * NCHUNK, KC)
  dstt = dstp.reshape(16, 2 * NCHUNK, KC)
  p = _make_agg(32, "col4")(hs1_flat, src4t, dstt, zz32)

  agg2, ssum, ssq = pl.pallas_call(
      _k4_body,
      grid=grid,
      in_specs=[_bspec((NB, 128), lambda i: (i, 0)),
                _bspec((NB, 128), lambda i: (i, 0)),
                _bspec((NB, 1), lambda i: (i, 0)),
                _bspec((4, 32, 256)), _bspec((1, 256))],
      out_specs=[_bspec((NB, 128), lambda i: (i, 0)),
                 _bspec((1, 256)), _bspec((1, 256))],
      out_shape=[jax.ShapeDtypeStruct((NP, 128), f32),
                 jax.ShapeDtypeStruct((1, 256), f32),
                 jax.ShapeDtypeStruct((1, 256), f32)],
  )(p, hs1, dinv, w2r, b2r)

  p16 = pl.pallas_call(
      _k5_body,
      grid=grid,
      in_specs=[_bspec((NB, 128), lambda i: (i, 0)),
                _bspec((1, 256)), _bspec((1, 256)),
                _bspec((4, 32, 256)), _bspec((1, 256)), _bspec((1, 256)),
                _bspec((1, 256)), _bspec((256, 1)),
                _bspec((NB, 1), lambda i: (i, 0))],
      out_specs=_bspec((NB, 16), lambda i: (i, 0)),
      out_shape=jax.ShapeDtypeStruct((NP, 16), f32),
  )(agg2, ssum, ssq, w2r, b2r, g2r, be2r, W3, dinv)

  t = _make_agg(16, "plain")(p16, srcp, dstp, zz16, ones16)

  o = pl.pallas_call(
      _k6_body,
      grid=grid,
      in_specs=[_bspec((2, NB, 16), lambda i: (0, i, 0)),
                _bspec((NB, 16), lambda i: (i, 0)),
                _bspec((NB, 1), lambda i: (i, 0)),
                _bspec((1, 1))],
      out_specs=_bspec((NB, 1), lambda i: (i, 0)),
      out_shape=jax.ShapeDtypeStruct((N, 1), f32),
  )(t, p16, dinv, b3r)

  return o
